# baseline (speedup 1.0000x reference)
import functools

import jax
import jax.numpy as jnp
from jax import lax
from jax.experimental import pallas as pl
from jax.experimental.pallas import tpu as pltpu


def _gates(xj, h, wih, whh, brz, bihn, bhhn):
    H = h.shape[-1]
    gi = jnp.dot(xj.astype(jnp.bfloat16), wih,
                 preferred_element_type=jnp.float32)
    gh = jnp.dot(h.astype(jnp.bfloat16), whh,
                 preferred_element_type=jnp.float32)
    rz = jnp.tanh(gi[:, 0:2 * H] + gh[:, 0:2 * H] + brz) * 0.5 + 0.5
    r = rz[:, 0:H]
    z = rz[:, H:2 * H]
    n = jnp.tanh(gi[:, 2 * H:] + bihn + r * (gh[:, 2 * H:] + bhhn))
    return n + z * (h - n)


def _bigru_chunk_kernel(seq_ref, x_hbm_ref,
                        wih_f_ref, whh_f_ref, brz_f_ref, bihn_f_ref,
                        bhhn_f_ref, wih_b_ref, whh_b_ref, brz_b_ref,
                        bihn_b_ref, bhhn_b_ref,
                        out_ref,
                        hf_ref, hb_ref, x_ref, of_ref, ob_ref,
                        x_sem, sf_sem, sb_sem,
                        *, t_chunk, unroll):
    i = pl.program_id(0)
    n_chunks = pl.num_programs(0)
    B, H = hf_ref.shape
    p = lax.rem(i, 2)
    half = (n_chunks + 1) // 2
    c_f = i
    c_b = n_chunks - 1 - i

    @pl.when(i == 0)
    def _():
        hf_ref[...] = jnp.zeros_like(hf_ref)
        hb_ref[...] = jnp.zeros_like(hb_ref)

    def x_copy(c):
        sl = pl.ds(c * (t_chunk * H), t_chunk * H)
        return pltpu.make_async_copy(x_hbm_ref.at[:, sl], x_ref.at[:, sl],
                                     x_sem.at[c])

    @pl.when(i == 0)
    def _():
        x_copy(0).start()
        if n_chunks > 1:
            x_copy(n_chunks - 1).start()

    @pl.when(i + 1 < half)
    def _():
        x_copy(i + 1).start()

    @pl.when((i + 1 < half) & (n_chunks - 2 - i != i + 1))
    def _():
        x_copy(n_chunks - 2 - i).start()

    @pl.when(i < half)
    def _():
        x_copy(c_f).wait()

    @pl.when((i < half) & (c_b != c_f))
    def _():
        x_copy(c_b).wait()

    def out_copies(slot, chunk_f, chunk_b):
        cf = pltpu.make_async_copy(
            of_ref.at[slot],
            out_ref.at[pl.ds(chunk_f * t_chunk, t_chunk), :, pl.ds(0, H)],
            sf_sem.at[slot])
        cb = pltpu.make_async_copy(
            ob_ref.at[slot],
            out_ref.at[pl.ds(chunk_b * t_chunk, t_chunk), :, pl.ds(H, H)],
            sb_sem.at[slot])
        return cf, cb

    @pl.when(i >= 2)
    def _():
        cf, cb = out_copies(p, i - 2, n_chunks + 1 - i)
        cf.wait()
        cb.wait()

    seq = seq_ref[...]
    wih_f = wih_f_ref[...]
    whh_f = whh_f_ref[...]
    wih_b = wih_b_ref[...]
    whh_b = whh_b_ref[...]
    brz_f = brz_f_ref[...]
    bihn_f = bihn_f_ref[...]
    bhhn_f = bhhn_f_ref[...]
    brz_b = brz_b_ref[...]
    bihn_b = bihn_b_ref[...]
    bhhn_b = bhhn_b_ref[...]
    t0f = i * t_chunk
    t0b = c_b * t_chunk

    def body(j, carry):
        hf, hb = carry
        jr = t_chunk - 1 - j
        xf = x_ref[:, pl.ds((t0f + j) * H, H)]
        xb = x_ref[:, pl.ds((t0b + jr) * H, H)]
        hf_new = _gates(xf, hf, wih_f, whh_f, brz_f, bihn_f, bhhn_f)
        hb_new = _gates(xb, hb, wih_b, whh_b, brz_b, bihn_b, bhhn_b)
        mf = seq > (t0f + j)
        mb = seq > (t0b + jr)
        of_ref[p, j] = jnp.where(mf, hf_new, 0.0)
        ob_ref[p, jr] = jnp.where(mb, hb_new, 0.0)
        return (jnp.where(mf, hf_new, hf),
                jnp.where(mb, hb_new, hb))

    hf, hb = lax.fori_loop(0, t_chunk, body, (hf_ref[...], hb_ref[...]),
                           unroll=unroll)
    hf_ref[...] = hf
    hb_ref[...] = hb

    cf, cb = out_copies(p, i, c_b)
    cf.start()
    cb.start()

    if n_chunks > 1:
        @pl.when(i == n_chunks - 1)
        def _():
            cf2, cb2 = out_copies(1 - p, i - 1, n_chunks - i)
            cf2.wait()
            cb2.wait()

    @pl.when(i == n_chunks - 1)
    def _():
        cf3, cb3 = out_copies(p, i, c_b)
        cf3.wait()
        cb3.wait()


def _bigru(x_bth, seq_lengths, wih_f, whh_f, brz_f, bihn_f, bhhn_f,
           wih_b, whh_b, brz_b, bihn_b, bhhn_b, *, t_chunk=16, unroll=16):
    B, T, H = x_bth.shape
    t_chunk = max(1, min(t_chunk, T))
    if T % t_chunk or B % 8:
        t_chunk = 8 if T % 8 == 0 else 1
    n_chunks = T // t_chunk

    x2v = x_bth.reshape(B, T * H).astype(jnp.bfloat16)
    seq2d = seq_lengths.astype(jnp.int32).reshape(B, 1)

    const = lambda i: (0, 0)
    wspec = lambda a: pl.BlockSpec(a.shape, const)

    kernel_fn = functools.partial(_bigru_chunk_kernel, t_chunk=t_chunk,
                                  unroll=min(unroll, t_chunk))

    blk_bytes = t_chunk * B * H * 4
    vmem_bytes = int(min(B * T * H * 2
                         + 2 * 2 * blk_bytes
                         + 4 * H * 3 * H * 2
                         + 2 * B * H * 4 + (8 << 20),
                         56 << 20))

    out = pl.pallas_call(
        kernel_fn,
        out_shape=jax.ShapeDtypeStruct((T, B, 2 * H), jnp.float32),
        grid=(n_chunks,),
        in_specs=[
            pl.BlockSpec(seq2d.shape, const),
            pl.BlockSpec(memory_space=pl.ANY),
            wspec(wih_f), wspec(whh_f), wspec(brz_f), wspec(bihn_f),
            wspec(bhhn_f),
            wspec(wih_b), wspec(whh_b), wspec(brz_b), wspec(bihn_b),
            wspec(bhhn_b),
        ],
        out_specs=pl.BlockSpec(memory_space=pl.ANY),
        scratch_shapes=[
            pltpu.VMEM((B, H), jnp.float32),
            pltpu.VMEM((B, H), jnp.float32),
            pltpu.VMEM((B, T * H), jnp.bfloat16),
            pltpu.VMEM((2, t_chunk, B, H), jnp.float32),
            pltpu.VMEM((2, t_chunk, B, H), jnp.float32),
            pltpu.SemaphoreType.DMA((T // t_chunk,)),
            pltpu.SemaphoreType.DMA((2,)),
            pltpu.SemaphoreType.DMA((2,)),
        ],
        compiler_params=pltpu.CompilerParams(
            dimension_semantics=("arbitrary",),
            vmem_limit_bytes=vmem_bytes),
    )(seq2d, x2v, wih_f, whh_f, brz_f, bihn_f, bhhn_f,
      wih_b, whh_b, brz_b, bihn_b, bhhn_b)

    return out


def kernel(x_bth, seq_lengths, w_ih_f, w_hh_f, b_ih_f, b_hh_f,
           w_ih_b, w_hh_b, b_ih_b, b_hh_b, embedding, fc_w, fc_b):
    H = x_bth.shape[-1]
    scale = jnp.concatenate([jnp.full((1, 2 * H), 0.5, jnp.float32),
                             jnp.ones((1, H), jnp.float32)], axis=-1)
    prep_w = lambda w: (w * scale).astype(jnp.bfloat16)
    prep_rz = lambda bi, bh: (bi + bh)[:, :2 * H] * jnp.float32(0.5)
    return _bigru(
        x_bth, seq_lengths,
        prep_w(w_ih_f), prep_w(w_hh_f), prep_rz(b_ih_f, b_hh_f),
        b_ih_f[:, 2 * H:], b_hh_f[:, 2 * H:],
        prep_w(w_ih_b), prep_w(w_hh_b), prep_rz(b_ih_b, b_hh_b),
        b_ih_b[:, 2 * H:], b_hh_b[:, 2 * H:])


# speedup vs baseline: 1.2469x; 1.2469x over previous
import functools

import jax
import jax.numpy as jnp
from jax import lax
from jax.experimental import pallas as pl
from jax.experimental.pallas import tpu as pltpu


def _gates(xj, h, wih, whh, brz, bihn, bhhn):
    H = h.shape[-1]
    gi = jnp.dot(xj.astype(jnp.bfloat16), wih,
                 preferred_element_type=jnp.float32)
    gh = jnp.dot(h.astype(jnp.bfloat16), whh,
                 preferred_element_type=jnp.float32)
    rz = jnp.tanh(gi[:, 0:2 * H] + gh[:, 0:2 * H] + brz) * 0.5 + 0.5
    r = rz[:, 0:H]
    z = rz[:, H:2 * H]
    n = jnp.tanh(gi[:, 2 * H:] + bihn + r * (gh[:, 2 * H:] + bhhn))
    return n + z * (h - n)


def _bigru_chunk_kernel(seq_ref, xf_ref, xb_ref,
                        wih_f_ref, whh_f_ref, brz_f_ref, bihn_f_ref,
                        bhhn_f_ref, wih_b_ref, whh_b_ref, brz_b_ref,
                        bihn_b_ref, bhhn_b_ref,
                        out_ref,
                        hf_ref, hb_ref, of_ref, ob_ref, sf_sem, sb_sem,
                        *, t_chunk, unroll):
    i = pl.program_id(0)
    n_chunks = pl.num_programs(0)
    B, H = hf_ref.shape
    p = lax.rem(i, 2)

    @pl.when(i == 0)
    def _():
        hf_ref[...] = jnp.zeros_like(hf_ref)
        hb_ref[...] = jnp.zeros_like(hb_ref)

    def copies(slot, chunk_f, chunk_b):
        cf = pltpu.make_async_copy(
            of_ref.at[slot],
            out_ref.at[pl.ds(chunk_f * t_chunk, t_chunk), :, pl.ds(0, H)],
            sf_sem.at[slot])
        cb = pltpu.make_async_copy(
            ob_ref.at[slot],
            out_ref.at[pl.ds(chunk_b * t_chunk, t_chunk), :, pl.ds(H, H)],
            sb_sem.at[slot])
        return cf, cb

    @pl.when(i >= 2)
    def _():
        cf, cb = copies(p, i - 2, n_chunks + 1 - i)
        cf.wait()
        cb.wait()

    seq = seq_ref[...]
    wih_f = wih_f_ref[...]
    whh_f = whh_f_ref[...]
    wih_b = wih_b_ref[...]
    whh_b = whh_b_ref[...]
    brz_f = brz_f_ref[...]
    bihn_f = bihn_f_ref[...]
    bhhn_f = bhhn_f_ref[...]
    brz_b = brz_b_ref[...]
    bihn_b = bihn_b_ref[...]
    bhhn_b = bhhn_b_ref[...]
    t0f = i * t_chunk
    t0b = (n_chunks - 1 - i) * t_chunk

    def body(j, carry):
        hf, hb = carry
        jr = t_chunk - 1 - j
        xf = xf_ref[:, pl.ds(pl.multiple_of(j * H, H), H)]
        xb = xb_ref[:, pl.ds(pl.multiple_of(jr * H, H), H)]
        hf_new = _gates(xf, hf, wih_f, whh_f, brz_f, bihn_f, bhhn_f)
        hb_new = _gates(xb, hb, wih_b, whh_b, brz_b, bihn_b, bhhn_b)
        mf = seq > (t0f + j)
        mb = seq > (t0b + jr)
        of_ref[p, j] = jnp.where(mf, hf_new, 0.0)
        ob_ref[p, jr] = jnp.where(mb, hb_new, 0.0)
        return (jnp.where(mf, hf_new, hf),
                jnp.where(mb, hb_new, hb))

    hf, hb = lax.fori_loop(0, t_chunk, body, (hf_ref[...], hb_ref[...]),
                           unroll=unroll)
    hf_ref[...] = hf
    hb_ref[...] = hb

    cf, cb = copies(p, i, n_chunks - 1 - i)
    cf.start()
    cb.start()

    if n_chunks > 1:
        @pl.when(i == n_chunks - 1)
        def _():
            cf2, cb2 = copies(1 - p, i - 1, n_chunks - i)
            cf2.wait()
            cb2.wait()

    @pl.when(i == n_chunks - 1)
    def _():
        cf3, cb3 = copies(p, i, n_chunks - 1 - i)
        cf3.wait()
        cb3.wait()


def _bigru(x_bth, seq_lengths, wih_f, whh_f, brz_f, bihn_f, bhhn_f,
           wih_b, whh_b, brz_b, bihn_b, bhhn_b, *, t_chunk=16, unroll=16):
    B, T, H = x_bth.shape
    t_chunk = max(1, min(t_chunk, T))
    if T % t_chunk or B % 8:
        t_chunk = 8 if T % 8 == 0 else 1
    n_chunks = T // t_chunk

    x2v = x_bth.reshape(B, T * H)
    seq2d = seq_lengths.astype(jnp.int32).reshape(B, 1)

    const = lambda i: (0, 0)
    xf_spec = pl.BlockSpec((B, t_chunk * H), lambda i: (0, i))
    xb_spec = pl.BlockSpec((B, t_chunk * H), lambda i: (0, n_chunks - 1 - i))
    wspec = lambda a: pl.BlockSpec(a.shape, const)

    kernel_fn = functools.partial(_bigru_chunk_kernel, t_chunk=t_chunk,
                                  unroll=min(unroll, t_chunk))

    blk_bytes = t_chunk * B * H * 4
    vmem_bytes = int(min(4 * 2 * blk_bytes
                         + 4 * H * 3 * H * 2
                         + 2 * B * H * 4 + (16 << 20),
                         56 << 20))

    out = pl.pallas_call(
        kernel_fn,
        out_shape=jax.ShapeDtypeStruct((T, B, 2 * H), jnp.float32),
        grid=(n_chunks,),
        in_specs=[
            pl.BlockSpec(seq2d.shape, const),
            xf_spec, xb_spec,
            wspec(wih_f), wspec(whh_f), wspec(brz_f), wspec(bihn_f),
            wspec(bhhn_f),
            wspec(wih_b), wspec(whh_b), wspec(brz_b), wspec(bihn_b),
            wspec(bhhn_b),
        ],
        out_specs=pl.BlockSpec(memory_space=pl.ANY),
        scratch_shapes=[
            pltpu.VMEM((B, H), jnp.float32),
            pltpu.VMEM((B, H), jnp.float32),
            pltpu.VMEM((2, t_chunk, B, H), jnp.float32),
            pltpu.VMEM((2, t_chunk, B, H), jnp.float32),
            pltpu.SemaphoreType.DMA((2,)),
            pltpu.SemaphoreType.DMA((2,)),
        ],
        compiler_params=pltpu.CompilerParams(
            dimension_semantics=("arbitrary",),
            vmem_limit_bytes=vmem_bytes),
    )(seq2d, x2v, x2v, wih_f, whh_f, brz_f, bihn_f, bhhn_f,
      wih_b, whh_b, brz_b, bihn_b, bhhn_b)

    return out


def kernel(x_bth, seq_lengths, w_ih_f, w_hh_f, b_ih_f, b_hh_f,
           w_ih_b, w_hh_b, b_ih_b, b_hh_b, embedding, fc_w, fc_b):
    H = x_bth.shape[-1]
    scale = jnp.concatenate([jnp.full((1, 2 * H), 0.5, jnp.float32),
                             jnp.ones((1, H), jnp.float32)], axis=-1)
    prep_w = lambda w: (w * scale).astype(jnp.bfloat16)
    prep_rz = lambda bi, bh: (bi + bh)[:, :2 * H] * jnp.float32(0.5)
    return _bigru(
        x_bth, seq_lengths,
        prep_w(w_ih_f), prep_w(w_hh_f), prep_rz(b_ih_f, b_hh_f),
        b_ih_f[:, 2 * H:], b_hh_f[:, 2 * H:],
        prep_w(w_ih_b), prep_w(w_hh_b), prep_rz(b_ih_b, b_hh_b),
        b_ih_b[:, 2 * H:], b_hh_b[:, 2 * H:])
